# baseline (speedup 1.0000x reference)
import functools

import jax
import jax.numpy as jnp
from jax import lax
from jax.experimental import pallas as pl
from jax.experimental.pallas import tpu as pltpu
from jax.experimental.pallas import tpu_sc as plsc

_B = 64
_NW = 32
_CHUNK = 3136


def _p1_kernel(seg_ref, x_ref, gw_ref, gb_ref, gate_ref, mout_ref, invout_ref,
               m_ref, d_ref):
    k = pl.program_id(0)
    nb = pl.num_programs(0)

    @pl.when(k == 0)
    def _init():
        m_ref[...] = jnp.full_like(m_ref, -jnp.inf)
        d_ref[...] = jnp.zeros_like(d_ref)

    x = x_ref[...]
    seg = seg_ref[0]
    bn = x.shape[0]
    g_row = lax.dot_general(
        gw_ref[...], x, (((0,), (1,)), ((), ())),
        preferred_element_type=jnp.float32) + gb_ref[...]
    gate_ref[0] = g_row

    iota_b = lax.broadcasted_iota(jnp.int32, (_B, bn), 0)
    m_blk = jnp.where(iota_b == seg, g_row, -jnp.inf)
    bm = jnp.max(m_blk, axis=1, keepdims=True)
    m_old = m_ref[...]
    m_new = jnp.maximum(m_old, bm)
    m_safe = jnp.where(m_new == -jnp.inf, 0.0, m_new)
    scale = jnp.exp(m_old - m_safe)
    e_blk = jnp.exp(m_blk - m_safe)
    d_ref[...] = d_ref[...] * scale + jnp.sum(e_blk, axis=1, keepdims=True)
    m_ref[...] = m_new

    @pl.when(k == nb - 1)
    def _fin():
        m = m_ref[...]
        d = d_ref[...]
        mout_ref[...] = jnp.where(m == -jnp.inf, 0.0, m)
        invout_ref[...] = jnp.where(d > 0, 1.0 / d, 0.0)


def _vgather(vec, idx):
    return lax.gather(
        vec, idx[:, None],
        lax.GatherDimensionNumbers(
            offset_dims=(), collapsed_slice_dims=(0,), start_index_map=(0,)),
        slice_sizes=(1,),
        mode=lax.GatherScatterMode.PROMISE_IN_BOUNDS)


def _sc_alpha(gate_hbm, seg_hbm, m_hbm, inv_hbm, alpha_hbm,
              g_v, s_v, m_v, i_v, a_v):
    c = lax.axis_index("c")
    s = lax.axis_index("s")
    wid = s * 2 + c
    base = wid * _CHUNK
    pltpu.sync_copy(gate_hbm.at[pl.ds(base, _CHUNK)], g_v)
    pltpu.sync_copy(seg_hbm.at[pl.ds(base, _CHUNK)], s_v)
    pltpu.sync_copy(m_hbm, m_v)
    pltpu.sync_copy(inv_hbm, i_v)
    mreg = [m_v[t] for t in range(4)]
    ireg = [i_v[t] for t in range(4)]

    def body(j, carry):
        g = g_v[pl.ds(j * 16, 16)]
        si = s_v[pl.ds(j * 16, 16)]
        hi = lax.shift_right_logical(si, 4)
        lo = lax.bitwise_and(si, 15)
        mseg = jnp.zeros((16,), jnp.float32)
        iseg = jnp.zeros((16,), jnp.float32)
        for t in range(4):
            sel = hi == t
            mseg = jnp.where(sel, _vgather(mreg[t], lo), mseg)
            iseg = jnp.where(sel, _vgather(ireg[t], lo), iseg)
        a_v[pl.ds(j * 16, 16)] = jnp.exp(g - mseg) * iseg
        return carry

    lax.fori_loop(0, _CHUNK // 16, body, 0)
    pltpu.sync_copy(a_v, alpha_hbm.at[pl.ds(base, _CHUNK)])


def _p2_kernel(seg_ref, x_ref, alpha_ref, inv_ref, fw_ref, fb_ref,
               out_ref, acc_ref):
    k = pl.program_id(0)
    nb = pl.num_programs(0)

    @pl.when(k == 0)
    def _init():
        acc_ref[...] = jnp.zeros_like(acc_ref)

    x = x_ref[...]
    seg = seg_ref[0]
    alpha_row = alpha_ref[0]
    bn = x.shape[0]

    iota_b = lax.broadcasted_iota(jnp.int32, (_B, bn), 0)
    a_blk = jnp.where(iota_b == seg, alpha_row, 0.0)
    acc_ref[...] += jnp.dot(a_blk, x, preferred_element_type=jnp.float32)

    @pl.when(k == nb - 1)
    def _fin():
        out = jnp.dot(acc_ref[...], fw_ref[...],
                      preferred_element_type=jnp.float32) + fb_ref[...]
        out_ref[...] = jnp.where(inv_ref[...] > 0, out, 0.0)


def kernel(x, segment_ids, gate_W, gate_b, feat_W, feat_b):
    n, d = x.shape
    bn = 10000
    nb = n // bn
    npad = _NW * _CHUNK

    seg32 = segment_ids.astype(jnp.int32)
    seg3 = seg32.reshape(nb, 1, bn)
    gb = gate_b.astype(jnp.float32).reshape(1, 1)
    fb = feat_b.astype(jnp.float32).reshape(1, d)

    gate3, m_col, inv_col = pl.pallas_call(
        _p1_kernel,
        grid=(nb,),
        in_specs=[
            pl.BlockSpec((1, 1, bn), lambda k: (k, 0, 0)),
            pl.BlockSpec((bn, d), lambda k: (k, 0)),
            pl.BlockSpec((d, 1), lambda k: (0, 0)),
            pl.BlockSpec((1, 1), lambda k: (0, 0)),
        ],
        out_specs=[
            pl.BlockSpec((1, 1, bn), lambda k: (k, 0, 0)),
            pl.BlockSpec((_B, 1), lambda k: (0, 0)),
            pl.BlockSpec((_B, 1), lambda k: (0, 0)),
        ],
        out_shape=[
            jax.ShapeDtypeStruct((nb, 1, bn), jnp.float32),
            jax.ShapeDtypeStruct((_B, 1), jnp.float32),
            jax.ShapeDtypeStruct((_B, 1), jnp.float32),
        ],
        scratch_shapes=[
            pltpu.VMEM((_B, 1), jnp.float32),
            pltpu.VMEM((_B, 1), jnp.float32),
        ],
    )(seg3, x, gate_W, gb)

    gate_pad = jnp.concatenate(
        [gate3.reshape(n), jnp.zeros(npad - n, jnp.float32)])
    seg_pad = jnp.concatenate([seg32, jnp.zeros(npad - n, jnp.int32)])
    m_tab = m_col.reshape(4, 16)
    inv_tab = inv_col.reshape(4, 16)

    sc_fn = functools.partial(
        pl.kernel,
        mesh=plsc.VectorSubcoreMesh(core_axis_name="c", subcore_axis_name="s"),
        out_type=jax.ShapeDtypeStruct((npad,), jnp.float32),
        scratch_types=[
            pltpu.VMEM((_CHUNK,), jnp.float32),
            pltpu.VMEM((_CHUNK,), jnp.int32),
            pltpu.VMEM((4, 16), jnp.float32),
            pltpu.VMEM((4, 16), jnp.float32),
            pltpu.VMEM((_CHUNK,), jnp.float32),
        ],
    )(_sc_alpha)
    alpha_pad = sc_fn(gate_pad, seg_pad, m_tab, inv_tab)
    alpha3 = alpha_pad[:n].reshape(nb, 1, bn)

    out = pl.pallas_call(
        _p2_kernel,
        grid=(nb,),
        in_specs=[
            pl.BlockSpec((1, 1, bn), lambda k: (k, 0, 0)),
            pl.BlockSpec((bn, d), lambda k: (k, 0)),
            pl.BlockSpec((1, 1, bn), lambda k: (k, 0, 0)),
            pl.BlockSpec((_B, 1), lambda k: (0, 0)),
            pl.BlockSpec((d, d), lambda k: (0, 0)),
            pl.BlockSpec((1, d), lambda k: (0, 0)),
        ],
        out_specs=pl.BlockSpec((_B, d), lambda k: (0, 0)),
        out_shape=jax.ShapeDtypeStruct((_B, d), jnp.float32),
        scratch_shapes=[pltpu.VMEM((_B, d), jnp.float32)],
    )(seg3, x, alpha3, inv_col, feat_W, fb)
    return out


# speedup vs baseline: 2.3663x; 2.3663x over previous
import jax
import jax.numpy as jnp
from jax.experimental import pallas as pl
from jax.experimental.pallas import tpu as pltpu

_B = 64


def _gap_kernel(seg_ref, x_ref, gw_ref, gb_ref, fw_ref, fb_ref, out_ref,
                acc_ref, m_ref, d_ref):
    k = pl.program_id(0)
    nb = pl.num_programs(0)

    @pl.when(k == 0)
    def _init():
        acc_ref[...] = jnp.zeros_like(acc_ref)
        m_ref[...] = jnp.full_like(m_ref, -jnp.inf)
        d_ref[...] = jnp.zeros_like(d_ref)

    x = x_ref[...]
    seg = seg_ref[0]
    bn = x.shape[0]

    g_row = jax.lax.dot_general(
        gw_ref[...], x, (((0,), (1,)), ((), ())),
        preferred_element_type=jnp.float32) + gb_ref[...]

    iota_b = jax.lax.broadcasted_iota(jnp.int32, (_B, bn), 0)
    onehot = iota_b == seg
    m_blk = jnp.where(onehot, g_row, -jnp.inf)

    bm = jnp.max(m_blk, axis=1, keepdims=True)
    m_old = m_ref[...]
    m_new = jnp.maximum(m_old, bm)
    m_safe = jnp.where(m_new == -jnp.inf, 0.0, m_new)
    scale = jnp.exp(m_old - m_safe)
    e_blk = jnp.exp(m_blk - m_safe)

    d_ref[...] = d_ref[...] * scale + jnp.sum(e_blk, axis=1, keepdims=True)
    acc_ref[...] = acc_ref[...] * scale + jnp.dot(
        e_blk, x, preferred_element_type=jnp.float32)
    m_ref[...] = m_new

    @pl.when(k == nb - 1)
    def _fin():
        d = d_ref[...]
        inv = jnp.where(d > 0, 1.0 / d, 0.0)
        pooled = acc_ref[...] * inv
        out = jnp.dot(pooled, fw_ref[...],
                      preferred_element_type=jnp.float32) + fb_ref[...]
        out_ref[...] = jnp.where(d > 0, out, 0.0)


def kernel(x, segment_ids, gate_W, gate_b, feat_W, feat_b):
    n, d = x.shape
    bn = 10000
    while n % bn != 0 or bn % 8 != 0:
        bn //= 2
    nb = n // bn

    seg = segment_ids.astype(jnp.int32).reshape(nb, 1, bn)
    gb = gate_b.astype(jnp.float32).reshape(1, 1)
    fb = feat_b.astype(jnp.float32).reshape(1, d)

    out = pl.pallas_call(
        _gap_kernel,
        grid=(nb,),
        in_specs=[
            pl.BlockSpec((1, 1, bn), lambda k: (k, 0, 0)),
            pl.BlockSpec((bn, d), lambda k: (k, 0)),
            pl.BlockSpec((d, 1), lambda k: (0, 0)),
            pl.BlockSpec((1, 1), lambda k: (0, 0)),
            pl.BlockSpec((d, d), lambda k: (0, 0)),
            pl.BlockSpec((1, d), lambda k: (0, 0)),
        ],
        out_specs=pl.BlockSpec((_B, d), lambda k: (0, 0)),
        out_shape=jax.ShapeDtypeStruct((_B, d), jnp.float32),
        scratch_shapes=[
            pltpu.VMEM((_B, d), jnp.float32),
            pltpu.VMEM((_B, 1), jnp.float32),
            pltpu.VMEM((_B, 1), jnp.float32),
        ],
    )(seg, x, gate_W, gb, feat_W, fb)
    return out
